# baseline (speedup 1.0000x reference)
import functools

import jax
import jax.numpy as jnp
from jax import lax
from jax.experimental import pallas as pl
from jax.experimental.pallas import tpu as pltpu
from jax.experimental.pallas import tpu_sc as plsc

N = 10000
D = 128
E = 320000

NC = 2
NS = 16
TILES = NC * NS
CHUNK = 128
CPT = -(-E // (TILES * CHUNK * 2)) * 2
EPT = CPT * CHUNK
EP = TILES * EPT
ACC_ROWS = 10240
ROWS_PT = ACC_ROWS // NS
LAST_ROWS = N - (NS - 1) * ROWS_PT
DUMMY = N
NBUF = 2
PASSES = 2
SLAB = CPT // PASSES


def _mm_body(x_ref, w_ref, b_ref, o_ref):
    o_ref[...] = (
        jnp.dot(x_ref[...], w_ref[...], preferred_element_type=jnp.float32)
        + b_ref[...]
    )


def _comb_body(h_ref, a_ref, o_ref):
    acc = a_ref[0] + a_ref[1]
    o_ref[...] = jnp.sum(h_ref[...] * acc, axis=1)


def _sc_body(h_hbm, src_hbm, dst_hbm, out_hbm,
             src_v, dst_v, buf_v, acc_sh, sem0, sem1):
    c = lax.axis_index("c")
    s = lax.axis_index("s")
    wid = c * NS + s
    sems = (sem0, sem1)

    zero16 = jnp.zeros((16,), jnp.float32)

    def zrow(r, carry):
        for l in range(D // 16):
            buf_v[0, r, pl.ds(l * 16, 16)] = zero16
        return carry

    lax.fori_loop(0, CHUNK, zrow, 0)

    def zcp(i, carry):
        pltpu.sync_copy(
            buf_v.at[0], acc_sh.at[pl.ds(s * ROWS_PT + i * CHUNK, CHUNK)])
        return carry

    lax.fori_loop(0, ROWS_PT // CHUNK, zcp, 0)
    plsc.subcore_barrier()

    for p in range(PASSES):
        pltpu.sync_copy(
            src_hbm.at[pl.ds(wid * CPT + p * SLAB, SLAB)], src_v)
        pltpu.sync_copy(
            dst_hbm.at[pl.ds(wid * CPT + p * SLAB, SLAB)], dst_v)

        for b_ in range(NBUF):
            pltpu.make_async_copy(
                h_hbm.at[src_v.at[b_]], buf_v.at[b_], sems[b_]).start()

        def pair(jo, carry):
            for b_ in range(NBUF):
                j = jo * NBUF + b_
                pltpu.make_async_copy(
                    h_hbm.at[src_v.at[j]], buf_v.at[b_], sems[b_]).wait()
                pltpu.sync_copy(buf_v.at[b_], acc_sh.at[dst_v.at[j]],
                                add=True)
                pltpu.make_async_copy(
                    h_hbm.at[src_v.at[j + NBUF]], buf_v.at[b_],
                    sems[b_]).start()
            return carry

        lax.fori_loop(0, (SLAB - NBUF) // NBUF, pair, 0)

        for b_ in range(NBUF):
            j = SLAB - NBUF + b_
            pltpu.make_async_copy(
                h_hbm.at[src_v.at[j]], buf_v.at[b_], sems[b_]).wait()
            pltpu.sync_copy(buf_v.at[b_], acc_sh.at[dst_v.at[j]], add=True)

    plsc.subcore_barrier()

    @pl.when(s < NS - 1)
    def _copy_full():
        pltpu.sync_copy(acc_sh.at[pl.ds(s * ROWS_PT, ROWS_PT)],
                        out_hbm.at[c].at[pl.ds(s * ROWS_PT, ROWS_PT)])

    @pl.when(s == NS - 1)
    def _copy_last():
        pltpu.sync_copy(acc_sh.at[pl.ds((NS - 1) * ROWS_PT, LAST_ROWS)],
                        out_hbm.at[c].at[pl.ds((NS - 1) * ROWS_PT, LAST_ROWS)])


@functools.cache
def _sc_agg():
    return pl.kernel(
        _sc_body,
        mesh=plsc.VectorSubcoreMesh(core_axis_name="c", subcore_axis_name="s"),
        out_type=jax.ShapeDtypeStruct((NC, N, D), jnp.float32),
        scratch_types=[
            pltpu.VMEM((SLAB, CHUNK), jnp.int32),
            pltpu.VMEM((SLAB, CHUNK), jnp.int32),
            pltpu.VMEM((NBUF, CHUNK, D), jnp.float32),
            pltpu.VMEM_SHARED((ACC_ROWS, D), jnp.float32),
            pltpu.SemaphoreType.DMA,
            pltpu.SemaphoreType.DMA,
        ],
    )


def kernel(x, edge_index, W, b):
    pad = EP - E
    pad_dst = DUMMY + (jnp.arange(pad, dtype=jnp.int32) % (ACC_ROWS - N))
    src = jnp.concatenate(
        [edge_index[0], jnp.zeros((pad,), jnp.int32)]).reshape(TILES * CPT, CHUNK)
    dst = jnp.concatenate(
        [edge_index[1], pad_dst]).reshape(TILES * CPT, CHUNK)

    h = pl.pallas_call(
        _mm_body,
        out_shape=jax.ShapeDtypeStruct((N, D), jnp.float32),
    )(x, W, b.reshape(1, D))

    agg = _sc_agg()(h, src, dst)

    scores = pl.pallas_call(
        _comb_body,
        out_shape=jax.ShapeDtypeStruct((N,), jnp.float32),
    )(h, agg)
    return scores


# speedup vs baseline: 1.0421x; 1.0421x over previous
import functools

import jax
import jax.numpy as jnp
from jax import lax
from jax.experimental import pallas as pl
from jax.experimental.pallas import tpu as pltpu
from jax.experimental.pallas import tpu_sc as plsc

N = 10000
D = 128
E = 320000

NC = 2
NS = 16
TILES = NC * NS
CHUNK = 128
NCHUNKS = -(-E // (TILES * CHUNK * 2)) * TILES * 2
EP = NCHUNKS * CHUNK
ACC_ROWS = 10240
ROWS_PT = ACC_ROWS // NS
LAST_ROWS = N - (NS - 1) * ROWS_PT
DUMMY = N
NBUF = 2
BIG_CPT = 128
SMALL_CPT = NCHUNKS // NS - BIG_CPT
ISLAB = 32


def _mm_body(x_ref, w_ref, b_ref, o_ref):
    o_ref[...] = (
        jnp.dot(x_ref[...], w_ref[...], preferred_element_type=jnp.float32)
        + b_ref[...]
    )


def _comb_body(h_ref, a_ref, o_ref):
    acc = a_ref[0] + a_ref[1]
    o_ref[...] = jnp.sum(h_ref[...] * acc, axis=1)


def _sc_body(h_hbm, src_hbm, dst_hbm, out_hbm,
             src_v, dst_v, buf_v, acc_sh, sem0, sem1):
    c = lax.axis_index("c")
    s = lax.axis_index("s")
    sems = (sem0, sem1)

    zero16 = jnp.zeros((16,), jnp.float32)

    def zrow(r, carry):
        for l in range(D // 16):
            buf_v[0, r, pl.ds(l * 16, 16)] = zero16
        return carry

    lax.fori_loop(0, CHUNK, zrow, 0)

    def zcp(i, carry):
        pltpu.sync_copy(
            buf_v.at[0], acc_sh.at[pl.ds(s * ROWS_PT + i * CHUNK, CHUNK)])
        return carry

    lax.fori_loop(0, ROWS_PT // CHUNK, zcp, 0)
    plsc.subcore_barrier()

    def run_edges(tile_chunk0, cpt):
        islab = ISLAB
        for p in range(cpt // islab):
            pltpu.sync_copy(
                src_hbm.at[pl.ds(tile_chunk0 + p * islab, islab)], src_v)
            pltpu.sync_copy(
                dst_hbm.at[pl.ds(tile_chunk0 + p * islab, islab)], dst_v)

            for b_ in range(NBUF):
                pltpu.make_async_copy(
                    h_hbm.at[src_v.at[b_]], buf_v.at[b_], sems[b_]).start()

            def pair(jo, carry):
                for b_ in range(NBUF):
                    j = jo * NBUF + b_
                    pltpu.make_async_copy(
                        h_hbm.at[src_v.at[j]], buf_v.at[b_], sems[b_]).wait()
                    pltpu.sync_copy(buf_v.at[b_], acc_sh.at[dst_v.at[j]],
                                    add=True)
                    pltpu.make_async_copy(
                        h_hbm.at[src_v.at[j + NBUF]], buf_v.at[b_],
                        sems[b_]).start()
                return carry

            lax.fori_loop(0, (islab - NBUF) // NBUF, pair, 0)

            for b_ in range(NBUF):
                j = islab - NBUF + b_
                pltpu.make_async_copy(
                    h_hbm.at[src_v.at[j]], buf_v.at[b_], sems[b_]).wait()
                pltpu.sync_copy(buf_v.at[b_], acc_sh.at[dst_v.at[j]],
                                add=True)

    @pl.when(c == 0)
    def _big_share():
        run_edges(s * BIG_CPT, BIG_CPT)

    @pl.when(c == 1)
    def _small_share():
        run_edges(NS * BIG_CPT + s * SMALL_CPT, SMALL_CPT)

    plsc.subcore_barrier()

    @pl.when(s < NS - 1)
    def _copy_full():
        pltpu.sync_copy(acc_sh.at[pl.ds(s * ROWS_PT, ROWS_PT)],
                        out_hbm.at[c].at[pl.ds(s * ROWS_PT, ROWS_PT)])

    @pl.when(s == NS - 1)
    def _copy_last():
        pltpu.sync_copy(acc_sh.at[pl.ds((NS - 1) * ROWS_PT, LAST_ROWS)],
                        out_hbm.at[c].at[pl.ds((NS - 1) * ROWS_PT, LAST_ROWS)])


@functools.cache
def _sc_agg():
    return pl.kernel(
        _sc_body,
        mesh=plsc.VectorSubcoreMesh(core_axis_name="c", subcore_axis_name="s"),
        out_type=jax.ShapeDtypeStruct((NC, N, D), jnp.float32),
        scratch_types=[
            pltpu.VMEM((ISLAB, CHUNK), jnp.int32),
            pltpu.VMEM((ISLAB, CHUNK), jnp.int32),
            pltpu.VMEM((NBUF, CHUNK, D), jnp.float32),
            pltpu.VMEM_SHARED((ACC_ROWS, D), jnp.float32),
            pltpu.SemaphoreType.DMA,
            pltpu.SemaphoreType.DMA,
        ],
    )


def kernel(x, edge_index, W, b):
    pad = EP - E
    pad_dst = DUMMY + (jnp.arange(pad, dtype=jnp.int32) % (ACC_ROWS - N))
    src = jnp.concatenate(
        [edge_index[0], jnp.zeros((pad,), jnp.int32)]).reshape(NCHUNKS, CHUNK)
    dst = jnp.concatenate(
        [edge_index[1], pad_dst]).reshape(NCHUNKS, CHUNK)

    h = pl.pallas_call(
        _mm_body,
        out_shape=jax.ShapeDtypeStruct((N, D), jnp.float32),
    )(x, W, b.reshape(1, D))

    agg = _sc_agg()(h, src, dst)

    scores = pl.pallas_call(
        _comb_body,
        out_shape=jax.ShapeDtypeStruct((N,), jnp.float32),
    )(h, agg)
    return scores


# speedup vs baseline: 2.9859x; 2.8654x over previous
import functools

import jax
import jax.numpy as jnp
from jax import lax
from jax.experimental import pallas as pl
from jax.experimental.pallas import tpu as pltpu
from jax.experimental.pallas import tpu_sc as plsc

N = 10000
D = 128
E = 320000

NC = 2
NS = 16
TILES = NC * NS
CHUNK = 128
NCHUNKS = -(-E // (TILES * CHUNK * 2)) * TILES * 2
EP = NCHUNKS * CHUNK
ACC_ROWS = 10240
ROWS_PT = ACC_ROWS // NS
LAST_ROWS = N - (NS - 1) * ROWS_PT
DUMMY = N
NBUF = 2
BIG_CPT = 80
SMALL_CPT = NCHUNKS // NS - BIG_CPT
ISLAB = 40


def _mm_body(x_ref, w_ref, b_ref, o_ref):
    o_ref[...] = (
        jnp.dot(x_ref[...], w_ref[...], preferred_element_type=jnp.float32)
        + b_ref[...]
    )


def _comb_body(h_ref, a_ref, o_ref):
    acc = a_ref[0] + a_ref[1]
    o_ref[...] = jnp.sum(h_ref[...] * acc, axis=1)


def _sc_body(h_hbm, src_hbm, dst_hbm, out_hbm,
             src_v, dst_v, buf_v, acc_sh, sem0, sem1):
    c = lax.axis_index("c")
    s = lax.axis_index("s")
    sems = (sem0, sem1)

    zero16 = jnp.zeros((16,), jnp.float32)

    def zrow(r, carry):
        for l in range(D // 16):
            buf_v[0, r, pl.ds(l * 16, 16)] = zero16
        return carry

    lax.fori_loop(0, CHUNK, zrow, 0)

    def zcp(i, carry):
        pltpu.sync_copy(
            buf_v.at[0], acc_sh.at[pl.ds(s * ROWS_PT + i * CHUNK, CHUNK)])
        return carry

    lax.fori_loop(0, ROWS_PT // CHUNK, zcp, 0)
    plsc.subcore_barrier()

    def run_edges(tile_chunk0, cpt):
        islab = ISLAB
        for p in range(cpt // islab):
            pltpu.sync_copy(
                src_hbm.at[pl.ds(tile_chunk0 + p * islab, islab)], src_v)
            pltpu.sync_copy(
                dst_hbm.at[pl.ds(tile_chunk0 + p * islab, islab)], dst_v)

            for b_ in range(NBUF):
                pltpu.make_async_copy(
                    h_hbm.at[src_v.at[b_]], buf_v.at[b_], sems[b_]).start()

            def pair(jo, carry):
                for b_ in range(NBUF):
                    j = jo * NBUF + b_
                    pltpu.make_async_copy(
                        h_hbm.at[src_v.at[j]], buf_v.at[b_], sems[b_]).wait()
                    pltpu.sync_copy(buf_v.at[b_], acc_sh.at[dst_v.at[j]],
                                    add=True)
                    pltpu.make_async_copy(
                        h_hbm.at[src_v.at[j + NBUF]], buf_v.at[b_],
                        sems[b_]).start()
                return carry

            lax.fori_loop(0, (islab - NBUF) // NBUF, pair, 0)

            for b_ in range(NBUF):
                j = islab - NBUF + b_
                pltpu.make_async_copy(
                    h_hbm.at[src_v.at[j]], buf_v.at[b_], sems[b_]).wait()
                pltpu.sync_copy(buf_v.at[b_], acc_sh.at[dst_v.at[j]],
                                add=True)

    @pl.when(c == 0)
    def _big_share():
        run_edges(s * BIG_CPT, BIG_CPT)

    @pl.when(c == 1)
    def _small_share():
        run_edges(NS * BIG_CPT + s * SMALL_CPT, SMALL_CPT)

    plsc.subcore_barrier()

    @pl.when(s < NS - 1)
    def _copy_full():
        pltpu.sync_copy(acc_sh.at[pl.ds(s * ROWS_PT, ROWS_PT)],
                        out_hbm.at[c].at[pl.ds(s * ROWS_PT, ROWS_PT)])

    @pl.when(s == NS - 1)
    def _copy_last():
        pltpu.sync_copy(acc_sh.at[pl.ds((NS - 1) * ROWS_PT, LAST_ROWS)],
                        out_hbm.at[c].at[pl.ds((NS - 1) * ROWS_PT, LAST_ROWS)])


@functools.cache
def _sc_agg():
    return pl.kernel(
        _sc_body,
        mesh=plsc.VectorSubcoreMesh(core_axis_name="c", subcore_axis_name="s"),
        out_type=jax.ShapeDtypeStruct((NC, N, D), jnp.float32),
        scratch_types=[
            pltpu.VMEM((ISLAB, CHUNK), jnp.int32),
            pltpu.VMEM((ISLAB, CHUNK), jnp.int32),
            pltpu.VMEM((NBUF, CHUNK, D), jnp.float32),
            pltpu.VMEM_SHARED((ACC_ROWS, D), jnp.float32),
            pltpu.SemaphoreType.DMA,
            pltpu.SemaphoreType.DMA,
        ],
    )


def kernel(x, edge_index, W, b):
    pad = EP - E
    pad_dst = DUMMY + (jnp.arange(pad, dtype=jnp.int32) % (ACC_ROWS - N))
    pad_src = jnp.arange(pad, dtype=jnp.int32) % N
    src = jnp.concatenate(
        [edge_index[0], pad_src]).reshape(NCHUNKS, CHUNK)
    dst = jnp.concatenate(
        [edge_index[1], pad_dst]).reshape(NCHUNKS, CHUNK)

    h = pl.pallas_call(
        _mm_body,
        out_shape=jax.ShapeDtypeStruct((N, D), jnp.float32),
    )(x, W, b.reshape(1, D))

    agg = _sc_agg()(h, src, dst)

    scores = pl.pallas_call(
        _comb_body,
        out_shape=jax.ShapeDtypeStruct((N,), jnp.float32),
    )(h, agg)
    return scores
